# baseline (speedup 1.0000x reference)
import functools

import jax
import jax.numpy as jnp
from jax.experimental import pallas as pl
from jax.experimental.pallas import tpu as pltpu

_PAD_VAL = 1e4


def _min_fold(qbf, kbf, half_ksq, local, cb):
    kb = kbf.shape[0]
    for c in range(kb // cb):
        kc = kbf if cb == kb else kbf[c * cb:(c + 1) * cb]
        dots = jax.lax.dot_general(
            qbf, kc, (((1,), (1,)), ((), ())),
            preferred_element_type=jnp.float32)
        for s in range(cb // 128):
            off = c * cb + s * 128
            ds = half_ksq[:, off:off + 128] - dots[:, s * 128:(s + 1) * 128]
            local = ds if local is None else jnp.minimum(local, ds)
    return local


def _half_ksq_row(k):
    halves = jnp.full((8, k.shape[1]), 0.5, jnp.float32)
    return jax.lax.dot_general(
        halves, k * k, (((1,), (1,)), ((), ())),
        preferred_element_type=jnp.float32)[:1]


def _nn_kernel(q_ref, k_ref, t_ref, o_ref, acc_ref, *, nk, cb):
    j = pl.program_id(1)
    q = q_ref[...]
    k = k_ref[...]
    qbf = q.astype(jnp.bfloat16)

    local = _min_fold(qbf, k.astype(jnp.bfloat16), _half_ksq_row(k),
                      None, cb)

    @pl.when(j == 0)
    def _():
        acc_ref[...] = local

    @pl.when(j > 0)
    def _():
        acc_ref[...] = jnp.minimum(acc_ref[...], local)

    @pl.when(j == nk - 1)
    def _():
        t = t_ref[...]
        acc = _min_fold(qbf, t.astype(jnp.bfloat16), _half_ksq_row(t),
                        acc_ref[...], t.shape[0])
        q_sq = jnp.sum(q * q, axis=1, keepdims=True)
        m = jnp.min(acc, axis=1, keepdims=True)
        o_ref[...] = jnp.sqrt(jnp.maximum(2.0 * m + q_sq, 0.0) + 1e-12)


def kernel(queries, keys):
    n_q, d_dim = queries.shape
    n_k = keys.shape[0]
    qb = 1024
    kb = 4096
    cb = 256
    nq = n_q // qb
    nk = n_k // kb
    n_tail = n_k - nk * kb
    tb = max(-(-n_tail // 128) * 128, 128)
    tail = jnp.pad(keys[nk * kb:], ((0, tb - n_tail), (0, 0)),
                   constant_values=_PAD_VAL)
    out = pl.pallas_call(
        functools.partial(_nn_kernel, nk=nk, cb=cb),
        grid=(nq, nk),
        in_specs=[
            pl.BlockSpec((qb, d_dim), lambda i, j: (i, 0)),
            pl.BlockSpec((kb, d_dim), lambda i, j: (j, 0)),
            pl.BlockSpec((tb, d_dim), lambda i, j: (0, 0)),
        ],
        out_specs=pl.BlockSpec((qb, 1), lambda i, j: (i, 0)),
        out_shape=jax.ShapeDtypeStruct((n_q, 1), jnp.float32),
        scratch_shapes=[pltpu.VMEM((qb, 128), jnp.float32)],
        compiler_params=pltpu.CompilerParams(
            dimension_semantics=("parallel", "arbitrary")),
    )(queries, keys, tail)
    return out[:, 0]


# speedup vs baseline: 1.0463x; 1.0463x over previous
import functools

import jax
import jax.numpy as jnp
from jax.experimental import pallas as pl
from jax.experimental.pallas import tpu as pltpu

_PAD_VAL = 1e4


def _min_fold(qbf, kbf, half_ksq, local, cb):
    kb = kbf.shape[0]
    for c in range(kb // cb):
        kc = kbf if cb == kb else kbf[c * cb:(c + 1) * cb]
        dots = jax.lax.dot_general(
            qbf, kc, (((1,), (1,)), ((), ())),
            preferred_element_type=jnp.float32)
        for s in range(cb // 128):
            off = c * cb + s * 128
            ds = half_ksq[:, off:off + 128] - dots[:, s * 128:(s + 1) * 128]
            local = ds if local is None else jnp.minimum(local, ds)
    return local


def _half_ksq_row(k):
    halves = jnp.full((8, k.shape[1]), 0.5, jnp.float32)
    return jax.lax.dot_general(
        halves, k * k, (((1,), (1,)), ((), ())),
        preferred_element_type=jnp.float32)[:1]


def _nn_kernel(q_ref, k_ref, t_ref, o_ref, acc_ref, *, nk, cb):
    j = pl.program_id(1)
    q = q_ref[...]
    k = k_ref[...]
    qbf = q.astype(jnp.bfloat16)

    local = _min_fold(qbf, k.astype(jnp.bfloat16), _half_ksq_row(k),
                      None, cb)

    @pl.when(j == 0)
    def _():
        acc_ref[...] = local

    @pl.when(j > 0)
    def _():
        acc_ref[...] = jnp.minimum(acc_ref[...], local)

    @pl.when(j == nk - 1)
    def _():
        t = t_ref[...]
        acc = _min_fold(qbf, t.astype(jnp.bfloat16), _half_ksq_row(t),
                        acc_ref[...], t.shape[0])
        q_sq = jnp.sum(q * q, axis=1, keepdims=True)
        m = jnp.min(acc, axis=1, keepdims=True)
        o_ref[...] = jnp.sqrt(jnp.maximum(2.0 * m + q_sq, 0.0) + 1e-12)


def kernel(queries, keys):
    n_q, d_dim = queries.shape
    n_k = keys.shape[0]
    qb = 1024
    kb = 16384
    cb = 256
    nq = n_q // qb
    nk = n_k // kb
    n_tail = n_k - nk * kb
    tb = max(-(-n_tail // 128) * 128, 128)
    tail = jnp.pad(keys[nk * kb:], ((0, tb - n_tail), (0, 0)),
                   constant_values=_PAD_VAL)
    out = pl.pallas_call(
        functools.partial(_nn_kernel, nk=nk, cb=cb),
        grid=(nq, nk),
        in_specs=[
            pl.BlockSpec((qb, d_dim), lambda i, j: (i, 0)),
            pl.BlockSpec((kb, d_dim), lambda i, j: (j, 0)),
            pl.BlockSpec((tb, d_dim), lambda i, j: (0, 0)),
        ],
        out_specs=pl.BlockSpec((qb, 1), lambda i, j: (i, 0)),
        out_shape=jax.ShapeDtypeStruct((n_q, 1), jnp.float32),
        scratch_shapes=[pltpu.VMEM((qb, 128), jnp.float32)],
        compiler_params=pltpu.CompilerParams(
            dimension_semantics=("parallel", "arbitrary")),
    )(queries, keys, tail)
    return out[:, 0]
